# Initial kernel scaffold; baseline (speedup 1.0000x reference)
import jax
import jax.numpy as jnp
from jax.experimental import pallas as pl


def kernel(adj_indices, adj_values, user_table, item_table):
    raise NotImplementedError("write your pallas kernel here")


# speedup vs baseline: 3.0303x; 3.0303x over previous
import functools

import jax
import jax.numpy as jnp
from jax import lax
from jax.experimental import pallas as pl
from jax.experimental.pallas import tpu as pltpu
from jax.experimental.pallas import tpu_sc as plsc

N_USERS = 25000
N_ITEMS = 25000
N = N_USERS + N_ITEMS
NPAD = 50048
D = 64
HALF = 32
N_LAYERS = 3
E = 800000

NC = 2
NS = 16
L = 16

CHUNK = 128
NCHUNKS = E // CHUNK
BASE_CHUNKS = NCHUNKS // NS
EXTRA = NCHUNKS - BASE_CHUNKS * NS
ROWS_PER_TILE = NPAD // NS
ZROWS = 136


def _spmm3_body(xcat, rowh, colh, valh, out, acc, colv, colav, rowv, valv,
                rows, zbuf, gsem):
    c = lax.axis_index("c")
    s = lax.axis_index("s")

    def zfill(i, _):
        zbuf[i, pl.ds(0, L)] = jnp.zeros((L,), jnp.float32)
        zbuf[i, pl.ds(L, L)] = jnp.zeros((L,), jnp.float32)
        return 0

    lax.fori_loop(0, ZROWS, zfill, 0)

    acc_base = s * ROWS_PER_TILE

    def zero_acc_slice():
        def zcopy(k, _):
            pltpu.sync_copy(zbuf, acc.at[pl.ds(acc_base + k * ZROWS, ZROWS)])
            return 0
        lax.fori_loop(0, ROWS_PER_TILE // ZROWS, zcopy, 0)

    start = s * BASE_CHUNKS + jnp.minimum(s, EXTRA)
    nchunks = jnp.where(s < EXTRA, BASE_CHUNKS + 1, BASE_CHUNKS)

    def edge_loop(src, src_base):
        off_vec = jnp.full((L,), src_base, jnp.int32)

        def chunk_body(i, _):
            ebase = (start + i) * CHUNK
            pltpu.sync_copy(colh.at[pl.ds(ebase, CHUNK)], colv)
            pltpu.sync_copy(rowh.at[pl.ds(ebase, CHUNK)], rowv)
            pltpu.sync_copy(valh.at[pl.ds(ebase, CHUNK)], valv)
            for g in range(CHUNK // L):
                colav[pl.ds(g * L, L)] = colv[pl.ds(g * L, L)] + off_vec
            pltpu.async_copy(src.at[colav], rows, gsem).wait()
            for g in range(CHUNK // L):
                v16 = valv[pl.ds(g * L, L)]
                for e in range(L):
                    b = jnp.full((L,), v16[e], jnp.float32)
                    r = g * L + e
                    rows[r, pl.ds(0, L)] = rows[r, pl.ds(0, L)] * b
                    rows[r, pl.ds(L, L)] = rows[r, pl.ds(L, L)] * b
            pltpu.sync_copy(rows, acc.at[rowv], add=True)
            return 0

        lax.fori_loop(0, nchunks, chunk_body, 0)

    zero_acc_slice()
    plsc.subcore_barrier()

    for layer in range(N_LAYERS):
        if layer == 0:
            src = xcat
            src_base = c * NPAD
        else:
            src = out
            src_base = ((layer - 1) * NC + c) * NPAD
        edge_loop(src, src_base)
        plsc.subcore_barrier()
        dst_base = (layer * NC + c) * NPAD + acc_base
        pltpu.sync_copy(acc.at[pl.ds(acc_base, ROWS_PER_TILE)],
                        out.at[pl.ds(dst_base, ROWS_PER_TILE)])
        if layer < N_LAYERS - 1:
            zero_acc_slice()
            plsc.subcore_barrier()


@jax.jit
def _sc_spmm3(xcat, row, col, val):
    mesh = plsc.VectorSubcoreMesh(core_axis_name="c", subcore_axis_name="s",
                                  num_cores=NC, num_subcores=NS)
    f = pl.kernel(
        _spmm3_body,
        out_type=jax.ShapeDtypeStruct((N_LAYERS * NC * NPAD, HALF), jnp.float32),
        mesh=mesh,
        scratch_types=(
            pltpu.VMEM_SHARED((NPAD, HALF), jnp.float32),
            pltpu.VMEM((CHUNK,), jnp.int32),
            pltpu.VMEM((CHUNK,), jnp.int32),
            pltpu.VMEM((CHUNK,), jnp.int32),
            pltpu.VMEM((CHUNK,), jnp.float32),
            pltpu.VMEM((CHUNK, HALF), jnp.float32),
            pltpu.VMEM((ZROWS, HALF), jnp.float32),
            pltpu.SemaphoreType.DMA,
        ),
        compiler_params=pltpu.CompilerParams(use_tc_tiling_on_sc=False),
    )
    return f(xcat, row, col, val)


def kernel(adj_indices, adj_values, user_table, item_table):
    emb = jnp.concatenate([user_table, item_table], axis=0)
    pad = jnp.zeros((NPAD - N, HALF), jnp.float32)
    xcat = jnp.concatenate([emb[:, :HALF], pad, emb[:, HALF:], pad], axis=0)
    row = adj_indices[0].astype(jnp.int32)
    col = adj_indices[1].astype(jnp.int32)
    val = adj_values.astype(jnp.float32)
    y = _sc_spmm3(xcat, row, col, val)
    total = emb
    for layer in range(N_LAYERS):
        lo = (layer * NC + 0) * NPAD
        hi = (layer * NC + 1) * NPAD
        total = total + jnp.concatenate([y[lo:lo + N], y[hi:hi + N]], axis=1)
    mean = total * (1.0 / (N_LAYERS + 1))
    return (mean[:N_USERS], mean[N_USERS:])


# speedup vs baseline: 6.7904x; 2.2408x over previous
import jax
import jax.numpy as jnp
from jax import lax
from jax.experimental import pallas as pl
from jax.experimental.pallas import tpu as pltpu
from jax.experimental.pallas import tpu_sc as plsc

N_USERS = 25000
N_ITEMS = 25000
N = N_USERS + N_ITEMS
NPAD = 50048
D = 64
HALF = 32
N_LAYERS = 3
E = 800000

NC = 2
NS = 16
L = 16

CHUNK = 128
NCHUNKS_PAD = 6272
E_PAD = NCHUNKS_PAD * CHUNK
TILE_CHUNKS = NCHUNKS_PAD // NS
SUP = 14
NSUP = TILE_CHUNKS // SUP
ROWS_PER_TILE = NPAD // NS
ZROWS = 136


def _spmm3_body(xcat, rowh, colh, valh, out, acc, colv, rowv, valv,
                rows, zbuf, isem, gsem, ssem):
    c = lax.axis_index("c")
    s = lax.axis_index("s")

    def zfill(i, _):
        zbuf[i, pl.ds(0, L)] = jnp.zeros((L,), jnp.float32)
        zbuf[i, pl.ds(L, L)] = jnp.zeros((L,), jnp.float32)
        return 0

    lax.fori_loop(0, ZROWS, zfill, 0)

    acc_base = s * ROWS_PER_TILE

    def zero_acc_slice():
        def zcopy(k, _):
            pltpu.sync_copy(zbuf, acc.at[pl.ds(acc_base + k * ZROWS, ZROWS)])
            return 0
        lax.fori_loop(0, ROWS_PER_TILE // ZROWS, zcopy, 0)

    start = s * TILE_CHUNKS

    def idx_load(k, kb, sync):
        base = start + k * SUP
        for src_h, dst_v in ((colh, colv), (rowh, rowv), (valh, valv)):
            if sync:
                pltpu.sync_copy(src_h.at[pl.ds(base, SUP)], dst_v.at[kb])
            else:
                pltpu.async_copy(src_h.at[pl.ds(base, SUP)], dst_v.at[kb], isem)

    def idx_wait(k, kb):
        base = start + k * SUP
        for src_h, dst_v in ((colh, colv), (rowh, rowv), (valh, valv)):
            pltpu.make_async_copy(src_h.at[pl.ds(base, SUP)],
                                  dst_v.at[kb], isem).wait()

    def gather_start(src, kb, j, b):
        pltpu.async_copy(src.at[colv.at[kb, j]], rows.at[b], gsem)

    def gather_wait(src, kb, j, b):
        pltpu.make_async_copy(src.at[colv.at[kb, j]], rows.at[b], gsem).wait()

    def scatter_start(kb, j, b):
        pltpu.async_copy(rows.at[b], acc.at[rowv.at[kb, j]], ssem, add=True)

    def scatter_wait(kb, j, b):
        pltpu.make_async_copy(rows.at[b], acc.at[rowv.at[kb, j]], ssem).wait()

    def edge_loop(src, src_base):
        off_vec = jnp.full((L,), src_base, jnp.int32)

        def superchunk(k, _):
            kb = lax.rem(k, 2)
            @pl.when(k > 0)
            def _():
                idx_wait(k, kb)

            @pl.when(k < NSUP - 1)
            def _():
                idx_load(k + 1, 1 - kb, sync=False)

            def coff(j, _):
                for g in range(CHUNK // L):
                    colv[kb, j, pl.ds(g * L, L)] = (
                        colv[kb, j, pl.ds(g * L, L)] + off_vec)
                return 0
            lax.fori_loop(0, SUP, coff, 0)

            gather_start(src, kb, 0, 0)

            def chunk_body(j, _):
                b = lax.rem(j, 2)
                gather_wait(src, kb, j, b)

                @pl.when(j > 0)
                def _():
                    scatter_wait(kb, j - 1, 1 - b)

                @pl.when(j < SUP - 1)
                def _():
                    gather_start(src, kb, j + 1, 1 - b)

                for g in range(CHUNK // L):
                    v16 = valv[kb, j, pl.ds(g * L, L)]
                    for e in range(L):
                        bc = jnp.full((L,), v16[e], jnp.float32)
                        r = g * L + e
                        rows[b, r, pl.ds(0, L)] = rows[b, r, pl.ds(0, L)] * bc
                        rows[b, r, pl.ds(L, L)] = rows[b, r, pl.ds(L, L)] * bc

                scatter_start(kb, j, b)
                return 0

            lax.fori_loop(0, SUP, chunk_body, 0)
            scatter_wait(kb, SUP - 1, lax.rem(SUP - 1, 2))
            return 0

        idx_load(0, 0, sync=True)
        lax.fori_loop(0, NSUP, superchunk, 0)

    zero_acc_slice()
    plsc.subcore_barrier()

    for layer in range(N_LAYERS):
        if layer == 0:
            src = xcat
            src_base = c * NPAD
        else:
            src = out
            src_base = ((layer - 1) * NC + c) * NPAD
        edge_loop(src, src_base)
        plsc.subcore_barrier()
        dst_base = (layer * NC + c) * NPAD + acc_base
        pltpu.sync_copy(acc.at[pl.ds(acc_base, ROWS_PER_TILE)],
                        out.at[pl.ds(dst_base, ROWS_PER_TILE)])
        if layer < N_LAYERS - 1:
            zero_acc_slice()
            plsc.subcore_barrier()


@jax.jit
def _sc_spmm3(xcat, row, col, val):
    mesh = plsc.VectorSubcoreMesh(core_axis_name="c", subcore_axis_name="s",
                                  num_cores=NC, num_subcores=NS)
    f = pl.kernel(
        _spmm3_body,
        out_type=jax.ShapeDtypeStruct((N_LAYERS * NC * NPAD, HALF), jnp.float32),
        mesh=mesh,
        scratch_types=(
            pltpu.VMEM_SHARED((NPAD, HALF), jnp.float32),
            pltpu.VMEM((2, SUP, CHUNK), jnp.int32),
            pltpu.VMEM((2, SUP, CHUNK), jnp.int32),
            pltpu.VMEM((2, SUP, CHUNK), jnp.float32),
            pltpu.VMEM((2, CHUNK, HALF), jnp.float32),
            pltpu.VMEM((ZROWS, HALF), jnp.float32),
            pltpu.SemaphoreType.DMA,
            pltpu.SemaphoreType.DMA,
            pltpu.SemaphoreType.DMA,
        ),
        compiler_params=pltpu.CompilerParams(use_tc_tiling_on_sc=False),
    )
    return f(xcat, row, col, val)


def kernel(adj_indices, adj_values, user_table, item_table):
    emb = jnp.concatenate([user_table, item_table], axis=0)
    pad = jnp.zeros((NPAD - N, HALF), jnp.float32)
    xcat = jnp.concatenate([emb[:, :HALF], pad, emb[:, HALF:], pad], axis=0)
    epad = jnp.zeros((E_PAD - E,), jnp.int32)
    row = jnp.concatenate([adj_indices[0].astype(jnp.int32), epad])
    col = jnp.concatenate([adj_indices[1].astype(jnp.int32), epad])
    val = jnp.concatenate([adj_values.astype(jnp.float32),
                           jnp.zeros((E_PAD - E,), jnp.float32)])
    row = row.reshape(NCHUNKS_PAD, CHUNK)
    col = col.reshape(NCHUNKS_PAD, CHUNK)
    val = val.reshape(NCHUNKS_PAD, CHUNK)
    y = _sc_spmm3(xcat, row, col, val)
    total = emb
    for layer in range(N_LAYERS):
        lo = (layer * NC + 0) * NPAD
        hi = (layer * NC + 1) * NPAD
        total = total + jnp.concatenate([y[lo:lo + N], y[hi:hi + N]], axis=1)
    mean = total * (1.0 / (N_LAYERS + 1))
    return (mean[:N_USERS], mean[N_USERS:])


# speedup vs baseline: 8.9078x; 1.3118x over previous
import jax
import jax.numpy as jnp
from jax import lax
from jax.experimental import pallas as pl
from jax.experimental.pallas import tpu as pltpu
from jax.experimental.pallas import tpu_sc as plsc

N_USERS = 25000
N_ITEMS = 25000
N = N_USERS + N_ITEMS
NPAD = 50048
D = 64
HALF = 32
N_LAYERS = 3
E = 800000

NC = 2
NS = 16
L = 16

CHUNK = 128
NCHUNKS_PAD = 6272
E_PAD = NCHUNKS_PAD * CHUNK
TILE_CHUNKS = NCHUNKS_PAD // NS
SUP = 14
NSUP = TILE_CHUNKS // SUP
ROWS_PER_TILE = NPAD // NS


def _spmm3_body(xcat, rowh, colh, valh, zeros_h, out, acc, colv, rowv, valv,
                rows, isem, gsem, ssem):
    c = lax.axis_index("c")
    s = lax.axis_index("s")

    acc_base = s * ROWS_PER_TILE

    def zero_acc_slice():
        pltpu.sync_copy(zeros_h, acc.at[pl.ds(acc_base, ROWS_PER_TILE)])

    start = s * TILE_CHUNKS

    def idx_load(k, kb, sync):
        base = start + k * SUP
        for src_h, dst_v in ((colh, colv), (rowh, rowv), (valh, valv)):
            if sync:
                pltpu.sync_copy(src_h.at[pl.ds(base, SUP)], dst_v.at[kb])
            else:
                pltpu.async_copy(src_h.at[pl.ds(base, SUP)], dst_v.at[kb], isem)

    def idx_wait(k, kb):
        base = start + k * SUP
        for src_h, dst_v in ((colh, colv), (rowh, rowv), (valh, valv)):
            pltpu.make_async_copy(src_h.at[pl.ds(base, SUP)],
                                  dst_v.at[kb], isem).wait()

    def gather_start(src, kb, j, b):
        pltpu.async_copy(src.at[colv.at[kb, j]], rows.at[b], gsem)

    def gather_wait(src, kb, j, b):
        pltpu.make_async_copy(src.at[colv.at[kb, j]], rows.at[b], gsem).wait()

    def scatter_start(kb, j, b):
        pltpu.async_copy(rows.at[b], acc.at[rowv.at[kb, j]], ssem, add=True)

    def scatter_wait(kb, j, b):
        pltpu.make_async_copy(rows.at[b], acc.at[rowv.at[kb, j]], ssem).wait()

    def edge_loop(src, src_base):
        off_vec = jnp.full((L,), src_base, jnp.int32)

        def superchunk(k, _):
            kb = lax.rem(k, 2)
            @pl.when(k > 0)
            def _():
                idx_wait(k, kb)

            @pl.when(k < NSUP - 1)
            def _():
                idx_load(k + 1, 1 - kb, sync=False)

            def coff(j, _):
                for g in range(CHUNK // L):
                    colv[kb, j, pl.ds(g * L, L)] = (
                        colv[kb, j, pl.ds(g * L, L)] + off_vec)
                return 0
            lax.fori_loop(0, SUP, coff, 0)

            gather_start(src, kb, 0, 0)
            gather_start(src, kb, 1, 1)

            def chunk_body(j, _):
                b = lax.rem(j, 4)
                gather_wait(src, kb, j, b)

                @pl.when(j > 1)
                def _():
                    scatter_wait(kb, j - 2, lax.rem(j - 2, 4))

                @pl.when(j < SUP - 2)
                def _():
                    gather_start(src, kb, j + 2, lax.rem(j + 2, 4))

                for g in range(CHUNK // L):
                    v16 = valv[kb, j, pl.ds(g * L, L)]
                    for e in range(L):
                        bc = jnp.full((L,), v16[e], jnp.float32)
                        r = g * L + e
                        rows[b, r, pl.ds(0, L)] = rows[b, r, pl.ds(0, L)] * bc
                        rows[b, r, pl.ds(L, L)] = rows[b, r, pl.ds(L, L)] * bc

                scatter_start(kb, j, b)
                return 0

            lax.fori_loop(0, SUP, chunk_body, 0)
            scatter_wait(kb, SUP - 2, lax.rem(SUP - 2, 4))
            scatter_wait(kb, SUP - 1, lax.rem(SUP - 1, 4))
            return 0

        idx_load(0, 0, sync=True)
        lax.fori_loop(0, NSUP, superchunk, 0)

    zero_acc_slice()
    plsc.subcore_barrier()

    for layer in range(N_LAYERS):
        if layer == 0:
            src = xcat
            src_base = c * NPAD
        else:
            src = out
            src_base = ((layer - 1) * NC + c) * NPAD
        edge_loop(src, src_base)
        plsc.subcore_barrier()
        dst_base = (layer * NC + c) * NPAD + acc_base
        pltpu.sync_copy(acc.at[pl.ds(acc_base, ROWS_PER_TILE)],
                        out.at[pl.ds(dst_base, ROWS_PER_TILE)])
        if layer < N_LAYERS - 1:
            zero_acc_slice()
            plsc.subcore_barrier()


@jax.jit
def _sc_spmm3(xcat, row, col, val):
    mesh = plsc.VectorSubcoreMesh(core_axis_name="c", subcore_axis_name="s",
                                  num_cores=NC, num_subcores=NS)
    f = pl.kernel(
        _spmm3_body,
        out_type=jax.ShapeDtypeStruct((N_LAYERS * NC * NPAD, HALF), jnp.float32),
        mesh=mesh,
        scratch_types=(
            pltpu.VMEM_SHARED((NPAD, HALF), jnp.float32),
            pltpu.VMEM((2, SUP, CHUNK), jnp.int32),
            pltpu.VMEM((2, SUP, CHUNK), jnp.int32),
            pltpu.VMEM((2, SUP, CHUNK), jnp.float32),
            pltpu.VMEM((4, CHUNK, HALF), jnp.float32),
            pltpu.SemaphoreType.DMA,
            pltpu.SemaphoreType.DMA,
            pltpu.SemaphoreType.DMA,
        ),
        compiler_params=pltpu.CompilerParams(use_tc_tiling_on_sc=False),
    )
    zeros_h = jnp.zeros((ROWS_PER_TILE, HALF), jnp.float32)
    return f(xcat, row, col, val, zeros_h)


def kernel(adj_indices, adj_values, user_table, item_table):
    emb = jnp.concatenate([user_table, item_table], axis=0)
    pad = jnp.zeros((NPAD - N, HALF), jnp.float32)
    xcat = jnp.concatenate([emb[:, :HALF], pad, emb[:, HALF:], pad], axis=0)
    epad = jnp.zeros((E_PAD - E,), jnp.int32)
    row = jnp.concatenate([adj_indices[0].astype(jnp.int32), epad])
    col = jnp.concatenate([adj_indices[1].astype(jnp.int32), epad])
    val = jnp.concatenate([adj_values.astype(jnp.float32),
                           jnp.zeros((E_PAD - E,), jnp.float32)])
    row = row.reshape(NCHUNKS_PAD, CHUNK)
    col = col.reshape(NCHUNKS_PAD, CHUNK)
    val = val.reshape(NCHUNKS_PAD, CHUNK)
    y = _sc_spmm3(xcat, row, col, val)
    total = emb
    for layer in range(N_LAYERS):
        lo = (layer * NC + 0) * NPAD
        hi = (layer * NC + 1) * NPAD
        total = total + jnp.concatenate([y[lo:lo + N], y[hi:hi + N]], axis=1)
    mean = total * (1.0 / (N_LAYERS + 1))
    return (mean[:N_USERS], mean[N_USERS:])


# speedup vs baseline: 9.3766x; 1.0526x over previous
import jax
import jax.numpy as jnp
from jax import lax
from jax.experimental import pallas as pl
from jax.experimental.pallas import tpu as pltpu
from jax.experimental.pallas import tpu_sc as plsc

N_USERS = 25000
N_ITEMS = 25000
N = N_USERS + N_ITEMS
NPAD = 50048
D = 64
HALF = 32
N_LAYERS = 3
E = 800000

NC = 2
NS = 16
L = 16

CHUNK = 128
NCHUNKS = E // CHUNK
SUP = 10
SUPE = SUP * CHUNK
NSUPS = NCHUNKS // SUP
BASE_SUPS = NSUPS // NS
ROWS_PER_TILE = NPAD // NS
PIECE = 128
NPIECES = ROWS_PER_TILE // PIECE
PTAIL = ROWS_PER_TILE - NPIECES * PIECE


def _spmm3_body(rowh, colh, valh, emb2, zeros_h, ybuf, acc,
                colst, rowst, valst, colc, rowc, rows, isem, gsem, ssem):
    c = lax.axis_index("c")
    s = lax.axis_index("s")

    acc_base = s * ROWS_PER_TILE

    def zero_acc_slice():
        pltpu.sync_copy(zeros_h, acc.at[pl.ds(acc_base, ROWS_PER_TILE)])

    start = s * BASE_SUPS + jnp.minimum(s, NSUPS - BASE_SUPS * NS)
    nsup = jnp.where(s < NSUPS - BASE_SUPS * NS, BASE_SUPS + 1, BASE_SUPS)

    def idx_load(k, kb, sync):
        base = (start + k) * SUPE
        for src_h, dst_v in ((colh, colst), (rowh, rowst),
                             (valh, valst)):
            if sync:
                pltpu.sync_copy(src_h.at[pl.ds(base, SUPE)], dst_v.at[kb])
            else:
                pltpu.async_copy(src_h.at[pl.ds(base, SUPE)], dst_v.at[kb],
                                 isem)

    def idx_wait(k, kb):
        base = (start + k) * SUPE
        for src_h, dst_v in ((colh, colst), (rowh, rowst),
                             (valh, valst)):
            pltpu.make_async_copy(src_h.at[pl.ds(base, SUPE)],
                                  dst_v.at[kb], isem).wait()

    def gather_start(src, j, b):
        pltpu.async_copy(src.at[colc.at[b]], rows.at[b], gsem)

    def gather_wait(src, j, b):
        pltpu.make_async_copy(src.at[colc.at[b]], rows.at[b], gsem).wait()

    def scatter_start(b):
        pltpu.async_copy(rows.at[b], acc.at[rowc.at[b]], ssem, add=True)

    def scatter_wait(b):
        pltpu.make_async_copy(rows.at[b], acc.at[rowc.at[b]], ssem).wait()

    def edge_loop(src, src_base):
        off_vec = jnp.full((L,), src_base, jnp.int32)

        def fill_colc(kb, j, b):
            for g in range(CHUNK // L):
                colc[b, pl.ds(g * L, L)] = (
                    colst[kb, pl.ds(j * CHUNK + g * L, L)] + off_vec)

        def superchunk(k, _):
            kb = lax.rem(k, 2)
            @pl.when(k > 0)
            def _():
                idx_wait(k, kb)

            @pl.when(k < nsup - 1)
            def _():
                idx_load(k + 1, 1 - kb, sync=False)

            fill_colc(kb, 0, 0)
            gather_start(src, 0, 0)
            fill_colc(kb, 1, 1)
            gather_start(src, 1, 1)

            def chunk_body(j, _):
                b = lax.rem(j, 4)
                gather_wait(src, j, b)

                @pl.when(j > 1)
                def _():
                    scatter_wait(lax.rem(j - 2, 4))

                @pl.when(j < SUP - 2)
                def _():
                    b2 = lax.rem(j + 2, 4)
                    fill_colc(kb, j + 2, b2)
                    gather_start(src, j + 2, b2)

                for g in range(CHUNK // L):
                    rowc[b, pl.ds(g * L, L)] = (
                        rowst[kb, pl.ds(j * CHUNK + g * L, L)])

                for g in range(CHUNK // L):
                    v16 = valst[kb, pl.ds(j * CHUNK + g * L, L)]
                    for e in range(L):
                        bc = jnp.full((L,), v16[e], jnp.float32)
                        r = g * L + e
                        rows[b, r, pl.ds(0, L)] = rows[b, r, pl.ds(0, L)] * bc
                        rows[b, r, pl.ds(L, L)] = rows[b, r, pl.ds(L, L)] * bc

                scatter_start(b)
                return 0

            lax.fori_loop(0, SUP, chunk_body, 0)
            scatter_wait(lax.rem(SUP - 2, 4))
            scatter_wait(lax.rem(SUP - 1, 4))
            return 0

        idx_load(0, 0, sync=True)
        lax.fori_loop(0, nsup, superchunk, 0)

    zero_acc_slice()
    plsc.subcore_barrier()

    for layer in range(N_LAYERS):
        if layer == 0:
            src = emb2
            src_base = c * NPAD
        else:
            src = ybuf
            src_base = ((layer - 1) * NC + c) * NPAD
        edge_loop(src, src_base)
        plsc.subcore_barrier()
        dst_base = (layer * NC + c) * NPAD + acc_base
        pltpu.sync_copy(acc.at[pl.ds(acc_base, ROWS_PER_TILE)],
                        ybuf.at[pl.ds(dst_base, ROWS_PER_TILE)])
        if layer < N_LAYERS - 1:
            zero_acc_slice()
            plsc.subcore_barrier()


@jax.jit
def _sc_spmm3(row, col, val, emb2):
    mesh = plsc.VectorSubcoreMesh(core_axis_name="c", subcore_axis_name="s",
                                  num_cores=NC, num_subcores=NS)
    f = pl.kernel(
        _spmm3_body,
        out_type=jax.ShapeDtypeStruct((N_LAYERS * NC * NPAD, HALF),
                                      jnp.float32),
        mesh=mesh,
        scratch_types=(
            pltpu.VMEM_SHARED((NPAD, HALF), jnp.float32),
            pltpu.VMEM((2, SUPE), jnp.int32),
            pltpu.VMEM((2, SUPE), jnp.int32),
            pltpu.VMEM((2, SUPE), jnp.float32),
            pltpu.VMEM((4, CHUNK), jnp.int32),
            pltpu.VMEM((4, CHUNK), jnp.int32),
            pltpu.VMEM((4, CHUNK, HALF), jnp.float32),
            pltpu.SemaphoreType.DMA,
            pltpu.SemaphoreType.DMA,
            pltpu.SemaphoreType.DMA,
        ),
        compiler_params=pltpu.CompilerParams(use_tc_tiling_on_sc=False),
    )
    zeros_h = jnp.zeros((ROWS_PER_TILE, HALF), jnp.float32)
    return f(row, col, val, emb2, zeros_h)


def kernel(adj_indices, adj_values, user_table, item_table):
    emb = jnp.concatenate([user_table, item_table], axis=0)
    pad = jnp.zeros((NPAD - N, HALF), jnp.float32)
    emb2 = jnp.concatenate([emb[:, :HALF], pad, emb[:, HALF:], pad], axis=0)
    y = _sc_spmm3(adj_indices[0].astype(jnp.int32),
                  adj_indices[1].astype(jnp.int32),
                  adj_values.astype(jnp.float32), emb2)
    total = emb
    for layer in range(N_LAYERS):
        lo = (layer * NC + 0) * NPAD
        hi = (layer * NC + 1) * NPAD
        total = total + jnp.concatenate([y[lo:lo + N], y[hi:hi + N]], axis=1)
    mean = total * (1.0 / (N_LAYERS + 1))
    return (mean[:N_USERS], mean[N_USERS:])


# speedup vs baseline: 10.6271x; 1.1334x over previous
import jax
import jax.numpy as jnp
from jax import lax
from jax.experimental import pallas as pl
from jax.experimental.pallas import tpu as pltpu
from jax.experimental.pallas import tpu_sc as plsc

N_USERS = 25000
N_ITEMS = 25000
N = N_USERS + N_ITEMS
NPAD = 50048
D = 64
HALF = 32
N_LAYERS = 3
E = 800000

NC = 2
NS = 16
L = 16

CHUNK = 128
NCHUNKS = E // CHUNK
SUP = 10
SUPE = SUP * CHUNK
NSUPS = NCHUNKS // SUP
BASE_SUPS = NSUPS // NS
ROWS_PER_TILE = NPAD // NS
PIECE = 128
NPIECES = ROWS_PER_TILE // PIECE
PTAIL = ROWS_PER_TILE - NPIECES * PIECE


def _spmm3_body(rowh, colh, valh, emb2, zeros_h, ybuf, fin, acc,
                colst, rowst, valst, colc, rowc, rows, isem, gsem, ssem):
    c = lax.axis_index("c")
    s = lax.axis_index("s")

    acc_base = s * ROWS_PER_TILE

    def zero_acc_slice():
        pltpu.sync_copy(zeros_h, acc.at[pl.ds(acc_base, ROWS_PER_TILE)])

    start = s * BASE_SUPS + jnp.minimum(s, NSUPS - BASE_SUPS * NS)
    nsup = jnp.where(s < NSUPS - BASE_SUPS * NS, BASE_SUPS + 1, BASE_SUPS)

    def idx_load(k, kb, sync):
        base = (start + k) * SUPE
        for src_h, dst_v in ((colh, colst), (rowh, rowst),
                             (valh, valst)):
            if sync:
                pltpu.sync_copy(src_h.at[pl.ds(base, SUPE)], dst_v.at[kb])
            else:
                pltpu.async_copy(src_h.at[pl.ds(base, SUPE)], dst_v.at[kb],
                                 isem)

    def idx_wait(k, kb):
        base = (start + k) * SUPE
        for src_h, dst_v in ((colh, colst), (rowh, rowst),
                             (valh, valst)):
            pltpu.make_async_copy(src_h.at[pl.ds(base, SUPE)],
                                  dst_v.at[kb], isem).wait()

    def gather_start(src, j, b):
        pltpu.async_copy(src.at[colc.at[b]], rows.at[b], gsem)

    def gather_wait(src, j, b):
        pltpu.make_async_copy(src.at[colc.at[b]], rows.at[b], gsem).wait()

    def scatter_start(b):
        pltpu.async_copy(rows.at[b], acc.at[rowc.at[b]], ssem, add=True)

    def scatter_wait(b):
        pltpu.make_async_copy(rows.at[b], acc.at[rowc.at[b]], ssem).wait()

    def edge_loop(src, src_base):
        off_vec = jnp.full((L,), src_base, jnp.int32)

        def fill_colc(kb, j, b):
            for g in range(CHUNK // L):
                colc[b, pl.ds(g * L, L)] = (
                    colst[kb, pl.ds(j * CHUNK + g * L, L)] + off_vec)

        def superchunk(k, _):
            kb = lax.rem(k, 2)
            @pl.when(k > 0)
            def _():
                idx_wait(k, kb)

            @pl.when(k < nsup - 1)
            def _():
                idx_load(k + 1, 1 - kb, sync=False)

            fill_colc(kb, 0, 0)
            gather_start(src, 0, 0)
            fill_colc(kb, 1, 1)
            gather_start(src, 1, 1)

            def chunk_body(j, _):
                b = lax.rem(j, 4)
                gather_wait(src, j, b)

                @pl.when(j > 1)
                def _():
                    scatter_wait(lax.rem(j - 2, 4))

                @pl.when(j < SUP - 2)
                def _():
                    b2 = lax.rem(j + 2, 4)
                    fill_colc(kb, j + 2, b2)
                    gather_start(src, j + 2, b2)

                for g in range(CHUNK // L):
                    rowc[b, pl.ds(g * L, L)] = (
                        rowst[kb, pl.ds(j * CHUNK + g * L, L)])

                for g in range(CHUNK // L):
                    v16 = valst[kb, pl.ds(j * CHUNK + g * L, L)]
                    for e in range(L):
                        bc = jnp.full((L,), v16[e], jnp.float32)
                        r = g * L + e
                        rows[b, r, pl.ds(0, L)] = rows[b, r, pl.ds(0, L)] * bc
                        rows[b, r, pl.ds(L, L)] = rows[b, r, pl.ds(L, L)] * bc

                scatter_start(b)
                return 0

            lax.fori_loop(0, SUP, chunk_body, 0)
            scatter_wait(lax.rem(SUP - 2, 4))
            scatter_wait(lax.rem(SUP - 1, 4))
            return 0

        idx_load(0, 0, sync=True)
        lax.fori_loop(0, nsup, superchunk, 0)

    zero_acc_slice()
    plsc.subcore_barrier()

    for layer in range(N_LAYERS):
        if layer == 0:
            src = emb2
            src_base = c * NPAD
        else:
            src = ybuf
            src_base = ((layer - 1) * NC + c) * NPAD
        edge_loop(src, src_base)
        plsc.subcore_barrier()
        if layer < N_LAYERS - 1:
            dst_base = (layer * NC + c) * NPAD + acc_base
            pltpu.sync_copy(acc.at[pl.ds(acc_base, ROWS_PER_TILE)],
                            ybuf.at[pl.ds(dst_base, ROWS_PER_TILE)])
            zero_acc_slice()
            plsc.subcore_barrier()

    def mean_piece(pbase, prows):
        srcs = (emb2.at[pl.ds(c * NPAD + pbase, prows)],
                ybuf.at[pl.ds((0 * NC + c) * NPAD + pbase, prows)],
                ybuf.at[pl.ds((1 * NC + c) * NPAD + pbase, prows)],
                acc.at[pl.ds(pbase, prows)])
        for i, sr in enumerate(srcs):
            pltpu.sync_copy(sr, rows.at[i, pl.ds(0, prows)])
        quarter = jnp.full((L,), 0.25, jnp.float32)

        def prow(r, _):
            for h in range(2):
                d = pl.ds(h * L, L)
                rows[0, r, d] = (rows[0, r, d] + rows[1, r, d]
                                 + rows[2, r, d] + rows[3, r, d]) * quarter
            return 0
        lax.fori_loop(0, prows, prow, 0)
        pltpu.sync_copy(rows.at[0, pl.ds(0, prows)],
                        fin.at[pl.ds(c * NPAD + pbase, prows)])

    def piece_loop(p, _):
        mean_piece(acc_base + p * PIECE, PIECE)
        return 0

    lax.fori_loop(0, NPIECES, piece_loop, 0)
    mean_piece(acc_base + NPIECES * PIECE, PTAIL)


@jax.jit
def _sc_spmm3(row, col, val, emb2):
    mesh = plsc.VectorSubcoreMesh(core_axis_name="c", subcore_axis_name="s",
                                  num_cores=NC, num_subcores=NS)
    f = pl.kernel(
        _spmm3_body,
        out_type=(
            jax.ShapeDtypeStruct((2 * NC * NPAD, HALF), jnp.float32),
            jax.ShapeDtypeStruct((NC * NPAD, HALF), jnp.float32),
        ),
        mesh=mesh,
        scratch_types=(
            pltpu.VMEM_SHARED((NPAD, HALF), jnp.float32),
            pltpu.VMEM((2, SUPE), jnp.int32),
            pltpu.VMEM((2, SUPE), jnp.int32),
            pltpu.VMEM((2, SUPE), jnp.float32),
            pltpu.VMEM((4, CHUNK), jnp.int32),
            pltpu.VMEM((4, CHUNK), jnp.int32),
            pltpu.VMEM((4, CHUNK, HALF), jnp.float32),
            pltpu.SemaphoreType.DMA,
            pltpu.SemaphoreType.DMA,
            pltpu.SemaphoreType.DMA,
        ),
        compiler_params=pltpu.CompilerParams(use_tc_tiling_on_sc=False),
    )
    zeros_h = jnp.zeros((ROWS_PER_TILE, HALF), jnp.float32)
    _, fin = f(row, col, val, emb2, zeros_h)
    return fin


def kernel(adj_indices, adj_values, user_table, item_table):
    emb = jnp.concatenate([user_table, item_table], axis=0)
    pad = jnp.zeros((NPAD - N, HALF), jnp.float32)
    emb2 = jnp.concatenate([emb[:, :HALF], pad, emb[:, HALF:], pad], axis=0)
    fin = _sc_spmm3(adj_indices[0].astype(jnp.int32),
                    adj_indices[1].astype(jnp.int32),
                    adj_values.astype(jnp.float32), emb2)
    mean = jnp.concatenate([fin[:N], fin[NPAD:NPAD + N]], axis=1)
    return (mean[:N_USERS], mean[N_USERS:])


# speedup vs baseline: 11.9556x; 1.1250x over previous
import jax
import jax.numpy as jnp
from jax import lax
from jax.experimental import pallas as pl
from jax.experimental.pallas import tpu as pltpu
from jax.experimental.pallas import tpu_sc as plsc

N_USERS = 25000
N_ITEMS = 25000
N = N_USERS + N_ITEMS
NPAD = 50048
D = 64
HALF = 32
N_LAYERS = 3
E = 800000

NC = 2
NS = 16
L = 16

CHUNK = 128
NCHUNKS = E // CHUNK
BASE_CHUNKS = NCHUNKS // NS
EXTRA = NCHUNKS - BASE_CHUNKS * NS
RING = 6
ROWS_PER_TILE = NPAD // NS
PIECE = 128
NPIECES = ROWS_PER_TILE // PIECE
PTAIL = ROWS_PER_TILE - NPIECES * PIECE


def _spmm3_body(rowh, colh, valh, emb2, zeros_h, ybuf, fin, acc,
                colc, rowc, valc, rows, isem, gsem, ssem):
    c = lax.axis_index("c")
    s = lax.axis_index("s")

    acc_base = s * ROWS_PER_TILE

    def zero_acc_slice():
        pltpu.sync_copy(zeros_h, acc.at[pl.ds(acc_base, ROWS_PER_TILE)])

    start = s * BASE_CHUNKS + jnp.minimum(s, EXTRA)
    nch = jnp.where(s < EXTRA, BASE_CHUNKS + 1, BASE_CHUNKS)

    def idx_start(j):
        b = lax.rem(j, RING)
        base = (start + j) * CHUNK
        for src_h, dst_v in ((colh, colc), (rowh, rowc), (valh, valc)):
            pltpu.async_copy(src_h.at[pl.ds(base, CHUNK)], dst_v.at[b], isem)

    def idx_wait(j):
        b = lax.rem(j, RING)
        base = (start + j) * CHUNK
        for src_h, dst_v in ((colh, colc), (rowh, rowc), (valh, valc)):
            pltpu.make_async_copy(src_h.at[pl.ds(base, CHUNK)],
                                  dst_v.at[b], isem).wait()

    def gather_start(src, b):
        pltpu.async_copy(src.at[colc.at[b]], rows.at[b], gsem)

    def gather_wait(src, b):
        pltpu.make_async_copy(src.at[colc.at[b]], rows.at[b], gsem).wait()

    def scatter_start(b):
        pltpu.async_copy(rows.at[b], acc.at[rowc.at[b]], ssem, add=True)

    def scatter_wait(b):
        pltpu.make_async_copy(rows.at[b], acc.at[rowc.at[b]], ssem).wait()

    def edge_loop(src, src_base):
        off_vec = jnp.full((L,), src_base, jnp.int32)

        def offset_cols(j):
            b = lax.rem(j, RING)
            for g in range(CHUNK // L):
                colc[b, pl.ds(g * L, L)] = colc[b, pl.ds(g * L, L)] + off_vec

        for jj in range(4):
            idx_start(jj)
        for jj in range(2):
            idx_wait(jj)
            offset_cols(jj)
            gather_start(src, jj)

        def chunk_body(j, _):
            b = lax.rem(j, RING)
            gather_wait(src, b)

            @pl.when(j > 1)
            def _():
                scatter_wait(lax.rem(j - 2, RING))

            @pl.when(j + 2 < nch)
            def _():
                idx_wait(j + 2)
                offset_cols(j + 2)
                gather_start(src, lax.rem(j + 2, RING))

            @pl.when(j + 4 < nch)
            def _():
                idx_start(j + 4)

            for g in range(CHUNK // L):
                v16 = valc[b, pl.ds(g * L, L)]
                for e in range(L):
                    bc = jnp.full((L,), v16[e], jnp.float32)
                    r = g * L + e
                    rows[b, r, pl.ds(0, L)] = rows[b, r, pl.ds(0, L)] * bc
                    rows[b, r, pl.ds(L, L)] = rows[b, r, pl.ds(L, L)] * bc

            scatter_start(b)
            return 0

        lax.fori_loop(0, nch, chunk_body, 0)
        scatter_wait(lax.rem(nch - 2, RING))
        scatter_wait(lax.rem(nch - 1, RING))

    zero_acc_slice()
    plsc.subcore_barrier()

    for layer in range(N_LAYERS):
        if layer == 0:
            src = emb2
            src_base = c * NPAD
        else:
            src = ybuf
            src_base = ((layer - 1) * NC + c) * NPAD
        edge_loop(src, src_base)
        plsc.subcore_barrier()
        if layer < N_LAYERS - 1:
            dst_base = (layer * NC + c) * NPAD + acc_base
            pltpu.sync_copy(acc.at[pl.ds(acc_base, ROWS_PER_TILE)],
                            ybuf.at[pl.ds(dst_base, ROWS_PER_TILE)])
            zero_acc_slice()
            plsc.subcore_barrier()

    def mean_piece(pbase, prows):
        srcs = (emb2.at[pl.ds(c * NPAD + pbase, prows)],
                ybuf.at[pl.ds((0 * NC + c) * NPAD + pbase, prows)],
                ybuf.at[pl.ds((1 * NC + c) * NPAD + pbase, prows)],
                acc.at[pl.ds(pbase, prows)])
        for i, sr in enumerate(srcs):
            pltpu.sync_copy(sr, rows.at[i, pl.ds(0, prows)])
        quarter = jnp.full((L,), 0.25, jnp.float32)

        def prow(r, _):
            for h in range(2):
                d = pl.ds(h * L, L)
                rows[0, r, d] = (rows[0, r, d] + rows[1, r, d]
                                 + rows[2, r, d] + rows[3, r, d]) * quarter
            return 0
        lax.fori_loop(0, prows, prow, 0)
        pltpu.sync_copy(rows.at[0, pl.ds(0, prows)],
                        fin.at[pl.ds(c * NPAD + pbase, prows)])

    def piece_loop(p, _):
        mean_piece(acc_base + p * PIECE, PIECE)
        return 0

    lax.fori_loop(0, NPIECES, piece_loop, 0)
    mean_piece(acc_base + NPIECES * PIECE, PTAIL)


@jax.jit
def _sc_spmm3(row, col, val, emb2):
    mesh = plsc.VectorSubcoreMesh(core_axis_name="c", subcore_axis_name="s",
                                  num_cores=NC, num_subcores=NS)
    f = pl.kernel(
        _spmm3_body,
        out_type=(
            jax.ShapeDtypeStruct((2 * NC * NPAD, HALF), jnp.float32),
            jax.ShapeDtypeStruct((NC * NPAD, HALF), jnp.float32),
        ),
        mesh=mesh,
        scratch_types=(
            pltpu.VMEM_SHARED((NPAD, HALF), jnp.float32),
            pltpu.VMEM((RING, CHUNK), jnp.int32),
            pltpu.VMEM((RING, CHUNK), jnp.int32),
            pltpu.VMEM((RING, CHUNK), jnp.float32),
            pltpu.VMEM((RING, CHUNK, HALF), jnp.float32),
            pltpu.SemaphoreType.DMA,
            pltpu.SemaphoreType.DMA,
            pltpu.SemaphoreType.DMA,
        ),
        compiler_params=pltpu.CompilerParams(use_tc_tiling_on_sc=False),
    )
    zeros_h = jnp.zeros((ROWS_PER_TILE, HALF), jnp.float32)
    _, fin = f(row, col, val, emb2, zeros_h)
    return fin


def kernel(adj_indices, adj_values, user_table, item_table):
    emb = jnp.concatenate([user_table, item_table], axis=0)
    pad = jnp.zeros((NPAD - N, HALF), jnp.float32)
    emb2 = jnp.concatenate([emb[:, :HALF], pad, emb[:, HALF:], pad], axis=0)
    fin = _sc_spmm3(adj_indices[0].astype(jnp.int32),
                    adj_indices[1].astype(jnp.int32),
                    adj_values.astype(jnp.float32), emb2)
    mean = jnp.concatenate([fin[:N], fin[NPAD:NPAD + N]], axis=1)
    return (mean[:N_USERS], mean[N_USERS:])
